# initial kernel scaffold (unmeasured)
import jax
import jax.numpy as jnp
from jax import lax
from jax.experimental import pallas as pl
from jax.experimental.pallas import tpu as pltpu

M = 8192
D = 2048
ROW_BLK = 256


def _comm(partial):

    def body(p_ref, out_ref, send_sem, recv_sem):
        my_x = lax.axis_index("x")
        my_y = lax.axis_index("y")
        peer = (1 - my_x, my_y)

        barrier_sem = pltpu.get_barrier_semaphore()
        pl.semaphore_signal(
            barrier_sem, inc=1, device_id=peer,
            device_id_type=pl.DeviceIdType.MESH,
        )
        pl.semaphore_wait(barrier_sem, 1)

        rdma = pltpu.make_async_remote_copy(
            src_ref=p_ref.at[0],
            dst_ref=out_ref,
            send_sem=send_sem,
            recv_sem=recv_sem,
            device_id=peer,
            device_id_type=pl.DeviceIdType.MESH,
        )
        rdma.start()
        rdma.wait()

    return pl.pallas_call(
        body,
        out_shape=jax.ShapeDtypeStruct((M, D), jnp.float32),
        in_specs=[pl.BlockSpec(memory_space=pltpu.ANY)],
        out_specs=pl.BlockSpec(memory_space=pltpu.ANY),
        scratch_shapes=[
            pltpu.SemaphoreType.DMA,
            pltpu.SemaphoreType.DMA,
        ],
        compiler_params=pltpu.CompilerParams(
            collective_id=0, has_side_effects=True
        ),
    )(partial)


def _compute(mine, theirs, resid, gamma):
    def body(p_ref, t_ref, r_ref, g_ref, out_ref):
        y = p_ref[0] + t_ref[...] + r_ref[...]
        rms = jnp.sqrt(jnp.mean(y * y, axis=-1, keepdims=True) + 1e-6)
        out_ref[...] = y / rms * g_ref[...]

    grid = (M // ROW_BLK,)
    return pl.pallas_call(
        body,
        out_shape=jax.ShapeDtypeStruct((M, D), jnp.float32),
        grid=grid,
        in_specs=[
            pl.BlockSpec((1, ROW_BLK, D), lambda i: (0, i, 0)),
            pl.BlockSpec((ROW_BLK, D), lambda i: (i, 0)),
            pl.BlockSpec((ROW_BLK, D), lambda i: (i, 0)),
            pl.BlockSpec((D,), lambda i: (0,)),
        ],
        out_specs=pl.BlockSpec((ROW_BLK, D), lambda i: (i, 0)),
    )(mine, theirs, resid, gamma)


def kernel(partial, resid, gamma):
    theirs = _comm(partial)
    return _compute(partial, theirs, resid, gamma)


# baseline (device time: 806223 ns/iter reference)
import jax
import jax.numpy as jnp
from jax import lax
from jax.experimental import pallas as pl
from jax.experimental.pallas import tpu as pltpu

M = 8192
D = 2048
ROW_BLK = 256


def _comm(partial):

    def body(p_ref, out_ref, send_sem, recv_sem):
        my_x = lax.axis_index("x")
        my_y = lax.axis_index("y")
        peer = (1 - my_x, my_y)

        barrier_sem = pltpu.get_barrier_semaphore()
        pl.semaphore_signal(
            barrier_sem, inc=1, device_id=peer,
            device_id_type=pl.DeviceIdType.MESH,
        )
        pl.semaphore_wait(barrier_sem, 1)

        rdma = pltpu.make_async_remote_copy(
            src_ref=p_ref.at[0],
            dst_ref=out_ref,
            send_sem=send_sem,
            recv_sem=recv_sem,
            device_id=peer,
            device_id_type=pl.DeviceIdType.MESH,
        )
        rdma.start()
        rdma.wait()

    return pl.pallas_call(
        body,
        out_shape=jax.ShapeDtypeStruct((M, D), jnp.float32),
        in_specs=[pl.BlockSpec(memory_space=pl.ANY)],
        out_specs=pl.BlockSpec(memory_space=pl.ANY),
        scratch_shapes=[
            pltpu.SemaphoreType.DMA,
            pltpu.SemaphoreType.DMA,
        ],
        compiler_params=pltpu.CompilerParams(
            collective_id=0, has_side_effects=True
        ),
    )(partial)


def _compute(mine, theirs, resid, gamma):
    def body(p_ref, t_ref, r_ref, g_ref, out_ref):
        y = p_ref[0] + t_ref[...] + r_ref[...]
        rms = jnp.sqrt(jnp.mean(y * y, axis=-1, keepdims=True) + 1e-6)
        out_ref[...] = y / rms * g_ref[...]

    grid = (M // ROW_BLK,)
    return pl.pallas_call(
        body,
        out_shape=jax.ShapeDtypeStruct((M, D), jnp.float32),
        grid=grid,
        in_specs=[
            pl.BlockSpec((1, ROW_BLK, D), lambda i: (0, i, 0)),
            pl.BlockSpec((ROW_BLK, D), lambda i: (i, 0)),
            pl.BlockSpec((ROW_BLK, D), lambda i: (i, 0)),
            pl.BlockSpec((D,), lambda i: (0,)),
        ],
        out_specs=pl.BlockSpec((ROW_BLK, D), lambda i: (i, 0)),
    )(mine, theirs, resid, gamma)


def kernel(partial, resid, gamma):
    theirs = _comm(partial)
    return _compute(partial, theirs, resid, gamma)


# device time: 436677 ns/iter; 1.8463x vs baseline; 1.8463x over previous
import jax
import jax.numpy as jnp
from jax import lax
from jax.experimental import pallas as pl
from jax.experimental.pallas import tpu as pltpu

M = 8192
D = 2048
HALF = M // 2
NC = 16
CH = HALF // NC


def kernel(partial, resid, gamma):
    def body(p_ref, r_ref, g_ref, out_ref, xrecv,
             theirs, mine, rbuf, ybuf,
             x_send_sems, x_recv_sems, y_send_sems, y_recv_sems,
             theirs_sems, mine_sems, rbuf_sems, store_sems):
        my_x = lax.axis_index("x")
        my_y = lax.axis_index("y")
        xpeer = (1 - my_x, my_y)
        ypeer = (my_x, 1 - my_y)
        my_start = my_y * HALF

        barrier_sem = pltpu.get_barrier_semaphore()
        for peer in (xpeer, ypeer):
            pl.semaphore_signal(
                barrier_sem, inc=1, device_id=peer,
                device_id_type=pl.DeviceIdType.MESH,
            )
        pl.semaphore_wait(barrier_sem, 2)

        x_rdmas = []
        for c in range(NC):
            rdma = pltpu.make_async_remote_copy(
                src_ref=p_ref.at[0, pl.ds(my_start + c * CH, CH), :],
                dst_ref=xrecv.at[c],
                send_sem=x_send_sems.at[c],
                recv_sem=x_recv_sems.at[c],
                device_id=xpeer,
                device_id_type=pl.DeviceIdType.MESH,
            )
            rdma.start()
            x_rdmas.append(rdma)

        y_rdmas = [None] * NC
        for c in range(NC):
            s = c % 2
            m_cp = pltpu.make_async_copy(
                p_ref.at[0, pl.ds(my_start + c * CH, CH), :],
                mine.at[s], mine_sems.at[s])
            r_cp = pltpu.make_async_copy(
                r_ref.at[pl.ds(my_start + c * CH, CH), :],
                rbuf.at[s], rbuf_sems.at[s])
            m_cp.start()
            r_cp.start()
            x_rdmas[c].wait_recv()
            t_cp = pltpu.make_async_copy(
                xrecv.at[c], theirs.at[s], theirs_sems.at[s])
            t_cp.start()
            m_cp.wait()
            r_cp.wait()
            t_cp.wait()

            y = mine[s] + theirs[s] + rbuf[s]
            rms = jnp.sqrt(jnp.mean(y * y, axis=-1, keepdims=True) + 1e-6)
            ybuf[s] = y / rms * g_ref[...]

            rdma = pltpu.make_async_remote_copy(
                src_ref=ybuf.at[s],
                dst_ref=out_ref.at[pl.ds(my_start + c * CH, CH), :],
                send_sem=y_send_sems.at[c],
                recv_sem=y_recv_sems.at[c],
                device_id=ypeer,
                device_id_type=pl.DeviceIdType.MESH,
            )
            rdma.start()
            y_rdmas[c] = rdma

            cp = pltpu.make_async_copy(
                ybuf.at[s],
                out_ref.at[pl.ds(my_start + c * CH, CH), :],
                store_sems.at[s])
            cp.start()
            cp.wait()

        for c in range(NC):
            y_rdmas[c].wait_recv()
            y_rdmas[c].wait_send()
            x_rdmas[c].wait_send()

    out, _ = pl.pallas_call(
        body,
        out_shape=(
            jax.ShapeDtypeStruct((M, D), jnp.float32),
            jax.ShapeDtypeStruct((NC, CH, D), jnp.float32),
        ),
        in_specs=[
            pl.BlockSpec(memory_space=pl.ANY),
            pl.BlockSpec(memory_space=pl.ANY),
            pl.BlockSpec(memory_space=pltpu.VMEM),
        ],
        out_specs=(
            pl.BlockSpec(memory_space=pl.ANY),
            pl.BlockSpec(memory_space=pl.ANY),
        ),
        scratch_shapes=[
            pltpu.VMEM((2, CH, D), jnp.float32),
            pltpu.VMEM((2, CH, D), jnp.float32),
            pltpu.VMEM((2, CH, D), jnp.float32),
            pltpu.VMEM((2, CH, D), jnp.float32),
            pltpu.SemaphoreType.DMA((NC,)),
            pltpu.SemaphoreType.DMA((NC,)),
            pltpu.SemaphoreType.DMA((NC,)),
            pltpu.SemaphoreType.DMA((NC,)),
            pltpu.SemaphoreType.DMA((2,)),
            pltpu.SemaphoreType.DMA((2,)),
            pltpu.SemaphoreType.DMA((2,)),
            pltpu.SemaphoreType.DMA((2,)),
        ],
        compiler_params=pltpu.CompilerParams(
            collective_id=0, has_side_effects=True
        ),
    )(partial, resid, gamma)
    return out
